# Initial kernel scaffold; baseline (speedup 1.0000x reference)
import jax
import jax.numpy as jnp
from jax.experimental import pallas as pl


def kernel(lab_ids, subject_ids, lab_table, subject_table):
    raise NotImplementedError("write your pallas kernel here")


# speedup vs baseline: 1.5249x; 1.5249x over previous
import functools

import jax
import jax.numpy as jnp
from jax import lax
from jax.experimental import pallas as pl
from jax.experimental.pallas import tpu as pltpu
from jax.experimental.pallas import tpu_sc as plsc

D = 128
B = 16384
NC = 2
NS = 16
NW = NC * NS
BPW = B // NW
CH = 128
NCH = BPW // CH

_mesh = plsc.VectorSubcoreMesh(core_axis_name="c", subcore_axis_name="s")


@functools.partial(
    pl.kernel,
    mesh=_mesh,
    out_type=jax.ShapeDtypeStruct((B, 2 * D), jnp.float32),
    scratch_types=[
        pltpu.VMEM((NCH, CH), jnp.int32),
        pltpu.VMEM((NCH, CH), jnp.int32),
        pltpu.VMEM((CH, D), jnp.float32),
        pltpu.VMEM((CH, D), jnp.float32),
        pltpu.SemaphoreType.DMA,
        pltpu.SemaphoreType.DMA,
    ],
)
def _adapter(lab_ids3, sub_ids3, lab_table, sub_table, out,
             lidx, sidx, lrows, srows, lsem, ssem):
    wid = lax.axis_index("s") * NC + lax.axis_index("c")
    base = wid * BPW
    pltpu.sync_copy(lab_ids3.at[wid], lidx)
    pltpu.sync_copy(sub_ids3.at[wid], sidx)
    for j in range(NCH):
        r0 = base + j * CH
        pltpu.async_copy(lab_table.at[lidx.at[j]], lrows, lsem).wait()
        pltpu.sync_copy(lrows, out.at[pl.ds(r0, CH), pl.ds(0, D)])
        pltpu.async_copy(sub_table.at[sidx.at[j]], srows, ssem).wait()
        pltpu.sync_copy(srows, out.at[pl.ds(r0, CH), pl.ds(D, D)])


def kernel(lab_ids, subject_ids, lab_table, subject_table):
    lab3 = lab_ids.reshape(NW, NCH, CH)
    sub3 = subject_ids.reshape(NW, NCH, CH)
    return _adapter(lab3, sub3, lab_table, subject_table)
